# baseline (speedup 1.0000x reference)
import jax
import jax.numpy as jnp
from jax import lax
from jax.experimental import pallas as pl
from jax.experimental.pallas import tpu as pltpu
from jax.experimental.pallas import tpu_sc as plsc

N_NODES_IN = 10000
D = 128
E = 320000

NC = 1
NS = 16
NW = NC * NS
CHUNK = 128
NCH0 = 160
NCH1 = 0
NCHM = max(NCH0, NCH1)
E_PAD = NS * (NCH0 + NCH1) * CHUNK
NACC = 10240
ROWS_PER_TILE = NACC // NS
GRP = 4
NGRP0 = NCH0 // GRP
NGRP1 = NCH1 // GRP
ZROWS = 128
NZC = ROWS_PER_TILE // ZROWS
DDEG = 16


def _sc_body(feat, edges, part, dpart, acc, dacc, ebuf, obuf, rows0, rows1,
             sem0, sem1):
    c = lax.axis_index("c")
    s = lax.axis_index("s")
    w = c * NS + s
    rs = s * ROWS_PER_TILE

    z16 = jnp.zeros((16,), jnp.float32)

    def zrow(i, carry):
        for k in range(D // 16):
            rows0[i, pl.ds(k * 16, 16)] = z16
        obuf[i, :] = z16
        return carry

    lax.fori_loop(0, CHUNK, zrow, 0)
    for q in range(NZC):
        pltpu.sync_copy(rows0.at[pl.ds(0, ZROWS)],
                        acc.at[pl.ds(rs + q * ZROWS, ZROWS)])
        pltpu.sync_copy(obuf.at[pl.ds(0, ZROWS)],
                        dacc.at[pl.ds(rs + q * ZROWS, ZROWS)])

    o16 = jnp.ones((16,), jnp.float32)

    def orow(i, carry):
        obuf[i, :] = o16
        return carry

    lax.fori_loop(0, CHUNK, orow, 0)

    plsc.subcore_barrier()

    bufs = (rows0, rows1)
    sems = (sem0, sem1)

    def group(g, carry):
        pltpu.sync_copy(edges.at[w, pl.ds(g * GRP, GRP)], ebuf)
        pltpu.async_copy(feat.at[ebuf.at[0, 0]], rows0, sem0)
        for j in range(GRP):
            cur = bufs[j % 2]
            pltpu.make_async_copy(feat.at[ebuf.at[j, 0]], cur,
                                  sems[j % 2]).wait()
            if j + 1 < GRP:
                pltpu.async_copy(feat.at[ebuf.at[j + 1, 0]],
                                 bufs[(j + 1) % 2], sems[(j + 1) % 2])
            pltpu.sync_copy(cur, acc.at[ebuf.at[j, 1]], add=True)
            pltpu.sync_copy(obuf, dacc.at[ebuf.at[j, 1]], add=True)
        return carry

    lax.fori_loop(0, NGRP0, group, 0)

    plsc.subcore_barrier()

    pltpu.sync_copy(acc.at[pl.ds(rs, ROWS_PER_TILE)],
                    part.at[c, pl.ds(rs, ROWS_PER_TILE)])
    pltpu.sync_copy(dacc.at[pl.ds(rs, ROWS_PER_TILE)],
                    dpart.at[c, pl.ds(rs, ROWS_PER_TILE)])


@jax.jit
def _sc_aggregate(feat, edges):
    mesh = plsc.VectorSubcoreMesh(core_axis_name="c", subcore_axis_name="s",
                                  num_cores=NC, num_subcores=NS)
    return pl.kernel(
        _sc_body,
        out_type=[
            jax.ShapeDtypeStruct((NC, NACC, D), jnp.float32),
            jax.ShapeDtypeStruct((NC, NACC, DDEG), jnp.float32),
        ],
        mesh=mesh,
        scratch_types=[
            pltpu.VMEM_SHARED((NACC, D), jnp.float32),
            pltpu.VMEM_SHARED((NACC, DDEG), jnp.float32),
            pltpu.VMEM((GRP, 2, CHUNK), jnp.int32),
            pltpu.VMEM((CHUNK, DDEG), jnp.float32),
            pltpu.VMEM((CHUNK, D), jnp.float32),
            pltpu.VMEM((CHUNK, D), jnp.float32),
            pltpu.SemaphoreType.DMA,
            pltpu.SemaphoreType.DMA,
        ],
        compiler_params=pltpu.CompilerParams(use_tc_tiling_on_sc=False),
    )(feat, edges)


def _tc_body(part_ref, dpart_ref, w_ref, b_ref, out_ref):
    p = part_ref[0, :N_NODES_IN, :]
    deg = dpart_ref[0, :N_NODES_IN, 0:1]
    for cc in range(1, NC):
        p = p + part_ref[cc, :N_NODES_IN, :]
        deg = deg + dpart_ref[cc, :N_NODES_IN, 0:1]
    h = p / jnp.maximum(deg, 1.0)
    y = lax.dot_general(h, w_ref[...], (((1,), (1,)), ((), ())),
                        preferred_element_type=jnp.float32)
    out_ref[...] = y + b_ref[...]


@jax.jit
def _tc_finish(part, dpart, W, b2):
    return pl.pallas_call(
        _tc_body,
        out_shape=jax.ShapeDtypeStruct((N_NODES_IN, D), jnp.float32),
    )(part, dpart, W, b2)


def kernel(features, edge_index, W, b):
    src = edge_index[0]
    dst = edge_index[1]
    pad = E_PAD - E
    pad_dst = (N_NODES_IN + jnp.arange(pad, dtype=jnp.int32)
               % (NACC - N_NODES_IN)).astype(jnp.int32)
    srcp = jnp.concatenate([src, jnp.zeros((pad,), jnp.int32)])
    dstp = jnp.concatenate([dst, pad_dst])

    def layout(a):
        return a.reshape(NW, NCHM, CHUNK)

    edges = jnp.stack([layout(srcp), layout(dstp)], axis=2)
    part, dpart = _sc_aggregate(features, edges)
    return _tc_finish(part, dpart, W, b.reshape(1, D))


# speedup vs baseline: 1.4272x; 1.4272x over previous
import jax
import jax.numpy as jnp
from jax import lax
from jax.experimental import pallas as pl
from jax.experimental.pallas import tpu as pltpu
from jax.experimental.pallas import tpu_sc as plsc

N_NODES_IN = 10000
D = 128
E = 320000

NC = 2
NS = 16
NW = NC * NS
CHUNK = 128
NCH0 = 128
NCH1 = 32
NCHM = max(NCH0, NCH1)
E_PAD = NS * (NCH0 + NCH1) * CHUNK
NACC = 10240
ROWS_PER_TILE = NACC // NS
GRP = 4
NGRP0 = NCH0 // GRP
NGRP1 = NCH1 // GRP
ZROWS = 128
NZC = ROWS_PER_TILE // ZROWS
DDEG = 16


def _sc_body(feat, edges, part, dpart, acc, dacc, ebuf, obuf, rows0, rows1,
             sem0, sem1):
    c = lax.axis_index("c")
    s = lax.axis_index("s")
    w = c * NS + s
    rs = s * ROWS_PER_TILE

    z16 = jnp.zeros((16,), jnp.float32)

    def zrow(i, carry):
        for k in range(D // 16):
            rows0[i, pl.ds(k * 16, 16)] = z16
        obuf[i, :] = z16
        return carry

    lax.fori_loop(0, CHUNK, zrow, 0)
    for q in range(NZC):
        pltpu.sync_copy(rows0.at[pl.ds(0, ZROWS)],
                        acc.at[pl.ds(rs + q * ZROWS, ZROWS)])
        pltpu.sync_copy(obuf.at[pl.ds(0, ZROWS)],
                        dacc.at[pl.ds(rs + q * ZROWS, ZROWS)])

    o16 = jnp.ones((16,), jnp.float32)

    def orow(i, carry):
        obuf[i, :] = o16
        return carry

    lax.fori_loop(0, CHUNK, orow, 0)

    plsc.subcore_barrier()

    bufs = (rows0, rows1)
    sems = (sem0, sem1)

    def group(g, carry):
        pltpu.sync_copy(edges.at[w, pl.ds(g * GRP, GRP)], ebuf)
        pltpu.async_copy(feat.at[ebuf.at[0, 0]], rows0, sem0)
        for j in range(GRP):
            cur = bufs[j % 2]
            pltpu.make_async_copy(feat.at[ebuf.at[j, 0]], cur,
                                  sems[j % 2]).wait()
            if j + 1 < GRP:
                pltpu.async_copy(feat.at[ebuf.at[j + 1, 0]],
                                 bufs[(j + 1) % 2], sems[(j + 1) % 2])
            pltpu.sync_copy(cur, acc.at[ebuf.at[j, 1]], add=True)
            pltpu.sync_copy(obuf, dacc.at[ebuf.at[j, 1]], add=True)
        return carry

    ngrp = lax.select(c == 0, NGRP0, NGRP1)
    lax.fori_loop(0, ngrp, group, 0)

    plsc.subcore_barrier()

    pltpu.sync_copy(acc.at[pl.ds(rs, ROWS_PER_TILE)],
                    part.at[c, pl.ds(rs, ROWS_PER_TILE)])
    pltpu.sync_copy(dacc.at[pl.ds(rs, ROWS_PER_TILE)],
                    dpart.at[c, pl.ds(rs, ROWS_PER_TILE)])


@jax.jit
def _sc_aggregate(feat, edges):
    mesh = plsc.VectorSubcoreMesh(core_axis_name="c", subcore_axis_name="s",
                                  num_cores=NC, num_subcores=NS)
    return pl.kernel(
        _sc_body,
        out_type=[
            jax.ShapeDtypeStruct((NC, NACC, D), jnp.float32),
            jax.ShapeDtypeStruct((NC, NACC, DDEG), jnp.float32),
        ],
        mesh=mesh,
        scratch_types=[
            pltpu.VMEM_SHARED((NACC, D), jnp.float32),
            pltpu.VMEM_SHARED((NACC, DDEG), jnp.float32),
            pltpu.VMEM((GRP, 2, CHUNK), jnp.int32),
            pltpu.VMEM((CHUNK, DDEG), jnp.float32),
            pltpu.VMEM((CHUNK, D), jnp.float32),
            pltpu.VMEM((CHUNK, D), jnp.float32),
            pltpu.SemaphoreType.DMA,
            pltpu.SemaphoreType.DMA,
        ],
        compiler_params=pltpu.CompilerParams(use_tc_tiling_on_sc=False),
    )(feat, edges)


def _tc_body(part_ref, dpart_ref, w_ref, b_ref, out_ref):
    p = part_ref[0, :N_NODES_IN, :]
    deg = dpart_ref[0, :N_NODES_IN, 0:1]
    for cc in range(1, NC):
        p = p + part_ref[cc, :N_NODES_IN, :]
        deg = deg + dpart_ref[cc, :N_NODES_IN, 0:1]
    h = p / jnp.maximum(deg, 1.0)
    y = lax.dot_general(h, w_ref[...], (((1,), (1,)), ((), ())),
                        preferred_element_type=jnp.float32)
    out_ref[...] = y + b_ref[...]


@jax.jit
def _tc_finish(part, dpart, W, b2):
    return pl.pallas_call(
        _tc_body,
        out_shape=jax.ShapeDtypeStruct((N_NODES_IN, D), jnp.float32),
    )(part, dpart, W, b2)


def kernel(features, edge_index, W, b):
    src = edge_index[0]
    dst = edge_index[1]
    pad = E_PAD - E
    pad_dst = (N_NODES_IN + jnp.arange(pad, dtype=jnp.int32)
               % (NACC - N_NODES_IN)).astype(jnp.int32)
    srcp = jnp.concatenate([src, jnp.zeros((pad,), jnp.int32)])
    dstp = jnp.concatenate([dst, pad_dst])

    def layout(a):
        e0 = NS * NCH0 * CHUNK
        a0 = a[:e0].reshape(NS, NCH0, CHUNK)
        a1 = a[e0:].reshape(NS, NCH1, CHUNK)
        a0 = jnp.pad(a0, ((0, 0), (0, NCHM - NCH0), (0, 0)))
        a1 = jnp.pad(a1, ((0, 0), (0, NCHM - NCH1), (0, 0)))
        return jnp.concatenate([a0, a1], axis=0)

    edges = jnp.stack([layout(srcp), layout(dstp)], axis=2)
    part, dpart = _sc_aggregate(features, edges)
    return _tc_finish(part, dpart, W, b.reshape(1, D))


# speedup vs baseline: 1.8269x; 1.2800x over previous
import jax
import jax.numpy as jnp
from jax import lax
from jax.experimental import pallas as pl
from jax.experimental.pallas import tpu as pltpu
from jax.experimental.pallas import tpu_sc as plsc

N_NODES_IN = 10000
D = 128
E = 320000

NC = 2
NS = 16
NW = NC * NS
CHUNK = 128
NCH0 = 144
NCH1 = 16
NCHM = max(NCH0, NCH1)
E_PAD = NS * (NCH0 + NCH1) * CHUNK
NACC = 10240
ROWS_PER_TILE = NACC // NS
GRP = 4
NGRP0 = NCH0 // GRP
NGRP1 = NCH1 // GRP
ZROWS = 128
NZC = ROWS_PER_TILE // ZROWS
DDEG = 16


def _sc_body(feat, edges, part, dpart, acc, dacc, ebuf, obuf, rows0, rows1,
             sem0, sem1):
    c = lax.axis_index("c")
    s = lax.axis_index("s")
    w = c * NS + s
    rs = s * ROWS_PER_TILE

    z16 = jnp.zeros((16,), jnp.float32)

    def zrow(i, carry):
        for k in range(D // 16):
            rows0[i, pl.ds(k * 16, 16)] = z16
        obuf[i, :] = z16
        return carry

    lax.fori_loop(0, CHUNK, zrow, 0)
    for q in range(NZC):
        pltpu.sync_copy(rows0.at[pl.ds(0, ZROWS)],
                        acc.at[pl.ds(rs + q * ZROWS, ZROWS)])
        pltpu.sync_copy(obuf.at[pl.ds(0, ZROWS)],
                        dacc.at[pl.ds(rs + q * ZROWS, ZROWS)])

    o16 = jnp.ones((16,), jnp.float32)

    def orow(i, carry):
        obuf[i, :] = o16
        return carry

    lax.fori_loop(0, CHUNK, orow, 0)

    plsc.subcore_barrier()

    bufs = (rows0, rows1)
    sems = (sem0, sem1)

    def group(g, carry):
        pltpu.sync_copy(edges.at[w, pl.ds(g * GRP, GRP)], ebuf)
        pltpu.async_copy(feat.at[ebuf.at[0, 0]], rows0, sem0)
        for j in range(GRP):
            cur = bufs[j % 2]
            pltpu.make_async_copy(feat.at[ebuf.at[j, 0]], cur,
                                  sems[j % 2]).wait()
            if j + 1 < GRP:
                pltpu.async_copy(feat.at[ebuf.at[j + 1, 0]],
                                 bufs[(j + 1) % 2], sems[(j + 1) % 2])
            pltpu.sync_copy(cur, acc.at[ebuf.at[j, 1]], add=True)
            pltpu.sync_copy(obuf, dacc.at[ebuf.at[j, 1]], add=True)
        return carry

    ngrp = lax.select(c == 0, NGRP0, NGRP1)
    lax.fori_loop(0, ngrp, group, 0)

    plsc.subcore_barrier()

    pltpu.sync_copy(acc.at[pl.ds(rs, ROWS_PER_TILE)],
                    part.at[c, pl.ds(rs, ROWS_PER_TILE)])
    pltpu.sync_copy(dacc.at[pl.ds(rs, ROWS_PER_TILE)],
                    dpart.at[c, pl.ds(rs, ROWS_PER_TILE)])


@jax.jit
def _sc_aggregate(feat, edges):
    mesh = plsc.VectorSubcoreMesh(core_axis_name="c", subcore_axis_name="s",
                                  num_cores=NC, num_subcores=NS)
    return pl.kernel(
        _sc_body,
        out_type=[
            jax.ShapeDtypeStruct((NC, NACC, D), jnp.float32),
            jax.ShapeDtypeStruct((NC, NACC, DDEG), jnp.float32),
        ],
        mesh=mesh,
        scratch_types=[
            pltpu.VMEM_SHARED((NACC, D), jnp.float32),
            pltpu.VMEM_SHARED((NACC, DDEG), jnp.float32),
            pltpu.VMEM((GRP, 2, CHUNK), jnp.int32),
            pltpu.VMEM((CHUNK, DDEG), jnp.float32),
            pltpu.VMEM((CHUNK, D), jnp.float32),
            pltpu.VMEM((CHUNK, D), jnp.float32),
            pltpu.SemaphoreType.DMA,
            pltpu.SemaphoreType.DMA,
        ],
        compiler_params=pltpu.CompilerParams(use_tc_tiling_on_sc=False),
    )(feat, edges)


def _tc_body(part_ref, dpart_ref, w_ref, b_ref, out_ref):
    p = part_ref[0, :N_NODES_IN, :]
    deg = dpart_ref[0, :N_NODES_IN, 0:1]
    for cc in range(1, NC):
        p = p + part_ref[cc, :N_NODES_IN, :]
        deg = deg + dpart_ref[cc, :N_NODES_IN, 0:1]
    h = p / jnp.maximum(deg, 1.0)
    y = lax.dot_general(h, w_ref[...], (((1,), (1,)), ((), ())),
                        preferred_element_type=jnp.float32)
    out_ref[...] = y + b_ref[...]


@jax.jit
def _tc_finish(part, dpart, W, b2):
    return pl.pallas_call(
        _tc_body,
        out_shape=jax.ShapeDtypeStruct((N_NODES_IN, D), jnp.float32),
    )(part, dpart, W, b2)


def kernel(features, edge_index, W, b):
    src = edge_index[0]
    dst = edge_index[1]
    pad = E_PAD - E
    pad_dst = (N_NODES_IN + jnp.arange(pad, dtype=jnp.int32)
               % (NACC - N_NODES_IN)).astype(jnp.int32)
    srcp = jnp.concatenate([src, jnp.zeros((pad,), jnp.int32)])
    dstp = jnp.concatenate([dst, pad_dst])

    def layout(a):
        e0 = NS * NCH0 * CHUNK
        a0 = a[:e0].reshape(NS, NCH0, CHUNK)
        a1 = a[e0:].reshape(NS, NCH1, CHUNK)
        a0 = jnp.pad(a0, ((0, 0), (0, NCHM - NCH0), (0, 0)))
        a1 = jnp.pad(a1, ((0, 0), (0, NCHM - NCH1), (0, 0)))
        return jnp.concatenate([a0, a1], axis=0)

    edges = jnp.stack([layout(srcp), layout(dstp)], axis=2)
    part, dpart = _sc_aggregate(features, edges)
    return _tc_finish(part, dpart, W, b.reshape(1, D))
